# baseline (speedup 1.0000x reference)
import functools

import jax
import jax.numpy as jnp
from jax.experimental import pallas as pl
from jax.experimental.pallas import tpu as pltpu
from jax.experimental.pallas import tpu_sc as plsc

_EMBED = 128
_BATCH = 4096
_QUEUE = 65536
_TEMP = 0.07
_WOUT = _QUEUE + 1
_WPAD = 65664
_BM = 64
_NCHUNK = 4


def _mm_body(m_ref, nq_ref, t_ref, out_ref):
    out_ref[...] = jnp.dot(
        m_ref[...], nq_ref[...], preferred_element_type=jnp.float32)
    m32 = m_ref[...].astype(jnp.float32)
    t32 = t_ref[...].astype(jnp.float32)
    out_ref[:, 0:1] = jnp.sum(m32 * t32, axis=1, keepdims=True)


_NWORK = 32
_ROWS_PER_W = _EMBED // _NWORK
_TAIL = _QUEUE - _BATCH


def _sc_enqueue_body(q_hbm, tt_hbm, out_hbm, head_v, tail_v):
    wid = jax.lax.axis_index("s") * 2 + jax.lax.axis_index("c")

    def row(i, carry):
        r = wid * _ROWS_PER_W + i
        pltpu.sync_copy(tt_hbm.at[r], head_v)
        pltpu.sync_copy(head_v, out_hbm.at[r, pl.ds(0, _BATCH)])
        pltpu.sync_copy(q_hbm.at[r, pl.ds(_BATCH, _TAIL)], tail_v)
        pltpu.sync_copy(tail_v, out_hbm.at[r, pl.ds(_BATCH, _TAIL)])
        return carry

    jax.lax.fori_loop(0, _ROWS_PER_W, row, 0)


def _sc_enqueue(queue, teacher_t):
    mesh = plsc.VectorSubcoreMesh(core_axis_name="c", subcore_axis_name="s")
    kfn = functools.partial(
        pl.kernel,
        mesh=mesh,
        out_type=jax.ShapeDtypeStruct((_EMBED, _QUEUE), jnp.float32),
        scratch_types=[
            pltpu.VMEM((_BATCH,), jnp.float32),
            pltpu.VMEM((_TAIL,), jnp.float32),
        ],
    )(_sc_enqueue_body)
    return kfn(queue, teacher_t)


def kernel(emb_student, emb_teacher, queue):
    nq = jnp.concatenate(
        [jnp.zeros((_EMBED, 1), jnp.float32), queue,
         jnp.zeros((_EMBED, _WPAD - _WOUT), jnp.float32)],
        axis=1).astype(jnp.bfloat16)
    m_bf = (emb_student * (1.0 / _TEMP)).astype(jnp.bfloat16)
    t_bf = emb_teacher.astype(jnp.bfloat16)

    def _mm_chunk(m_c, t_c):
        rows = m_c.shape[0]
        return pl.pallas_call(
            _mm_body,
            grid=(rows // _BM,),
            in_specs=[
                pl.BlockSpec((_BM, _EMBED), lambda i: (i, 0)),
                pl.BlockSpec((_EMBED, _WPAD), lambda i: (0, 0)),
                pl.BlockSpec((_BM, _EMBED), lambda i: (i, 0)),
            ],
            out_specs=pl.BlockSpec((_BM, _WPAD), lambda i: (i, 0)),
            out_shape=jax.ShapeDtypeStruct((rows, _WPAD), jnp.float32),
            compiler_params=pltpu.CompilerParams(
                dimension_semantics=("arbitrary",),
            ),
        )(m_c, nq, t_c)

    chunk = _BATCH // _NCHUNK
    logits = jnp.zeros((_BATCH, _WOUT), jnp.float32)
    for c in range(_NCHUNK):
        pad_c = _mm_chunk(
            jax.lax.slice(m_bf, (c * chunk, 0), ((c + 1) * chunk, _EMBED)),
            jax.lax.slice(t_bf, (c * chunk, 0), ((c + 1) * chunk, _EMBED)))
        logits = jax.lax.dynamic_update_slice(
            logits, jax.lax.slice(pad_c, (0, 0), (chunk, _WOUT)),
            (c * chunk, 0))

    teacher_t = emb_teacher.T
    new_queue = _sc_enqueue(queue, teacher_t)

    labels = jnp.zeros((_BATCH,), dtype=jnp.int32)
    return (logits, labels, new_queue)


# speedup vs baseline: 1.7890x; 1.7890x over previous
import functools

import jax
import jax.numpy as jnp
from jax.experimental import pallas as pl
from jax.experimental.pallas import tpu as pltpu
from jax.experimental.pallas import tpu_sc as plsc

_EMBED = 128
_BATCH = 4096
_QUEUE = 65536
_TEMP = 0.07
_WOUT = _QUEUE + 1
_WPAD = 65664
_BM = 64


def _mm_body(m_ref, nq_ref, t_ref, out_ref):
    m_bf = (m_ref[...] * (1.0 / _TEMP)).astype(jnp.bfloat16)
    out_ref[...] = jnp.dot(
        m_bf, nq_ref[...], preferred_element_type=jnp.float32)
    lpos = jnp.sum(m_ref[...] * t_ref[...], axis=1, keepdims=True)
    out_ref[:, 0:1] = lpos * (1.0 / _TEMP)


_NWORK = 32
_ROWS_PER_W = _EMBED // _NWORK
_TAIL = _QUEUE - _BATCH


def _sc_enqueue_body(q_hbm, tt_hbm, out_hbm, head_v, tail_v):
    wid = jax.lax.axis_index("s") * 2 + jax.lax.axis_index("c")

    def row(i, carry):
        r = wid * _ROWS_PER_W + i
        pltpu.sync_copy(tt_hbm.at[r], head_v)
        pltpu.sync_copy(head_v, out_hbm.at[r, pl.ds(0, _BATCH)])
        pltpu.sync_copy(q_hbm.at[r, pl.ds(_BATCH, _TAIL)], tail_v)
        pltpu.sync_copy(tail_v, out_hbm.at[r, pl.ds(_BATCH, _TAIL)])
        return carry

    jax.lax.fori_loop(0, _ROWS_PER_W, row, 0)


def _sc_enqueue(queue, teacher_t):
    mesh = plsc.VectorSubcoreMesh(core_axis_name="c", subcore_axis_name="s")
    kfn = functools.partial(
        pl.kernel,
        mesh=mesh,
        out_type=jax.ShapeDtypeStruct((_EMBED, _QUEUE), jnp.float32),
        scratch_types=[
            pltpu.VMEM((_BATCH,), jnp.float32),
            pltpu.VMEM((_TAIL,), jnp.float32),
        ],
    )(_sc_enqueue_body)
    return kfn(queue, teacher_t)


def kernel(emb_student, emb_teacher, queue):
    nq = jnp.concatenate(
        [jnp.zeros((_EMBED, 1), jnp.float32), queue,
         jnp.zeros((_EMBED, _WPAD - _WOUT), jnp.float32)],
        axis=1).astype(jnp.bfloat16)

    padded = pl.pallas_call(
        _mm_body,
        grid=(_BATCH // _BM,),
        in_specs=[
            pl.BlockSpec((_BM, _EMBED), lambda i: (i, 0)),
            pl.BlockSpec((_EMBED, _WPAD), lambda i: (0, 0)),
            pl.BlockSpec((_BM, _EMBED), lambda i: (i, 0)),
        ],
        out_specs=pl.BlockSpec((_BM, _WPAD), lambda i: (i, 0)),
        out_shape=jax.ShapeDtypeStruct((_BATCH, _WPAD), jnp.float32),
        compiler_params=pltpu.CompilerParams(
            dimension_semantics=("arbitrary",),
        ),
    )(emb_student, nq, emb_teacher)

    logits = jax.lax.slice(padded, (0, 0), (_BATCH, _WOUT))

    teacher_t = emb_teacher.T
    new_queue = _sc_enqueue(queue, teacher_t)

    labels = jnp.zeros((_BATCH,), dtype=jnp.int32)
    return (logits, labels, new_queue)
